# baseline (speedup 1.0000x reference)
import functools

import jax
import jax.numpy as jnp
from jax import lax
from jax.experimental import pallas as pl
from jax.experimental.pallas import tpu as pltpu
from jax.experimental.pallas import tpu_sc as plsc

N = 20000
NPAD = 20480
NCELL = 729
CPAD = 768
CCOLS = 768
K = 8
M = 16
NNB = 26
NSIDE = 9
NVREG = (NNB * K) // 16
BIG = 1 << 30
NW = 32
APT = NPAD // NW


ROWS = 96


def _select_topk_fast(d, col, nvalid, npass, idx_bits):
    half = float(2 ** idx_bits)
    key = jnp.where(col < nvalid,
                    d * half + ((half - 1.0) - col.astype(jnp.float32)),
                    -1.0)
    idxs = []
    for _ in range(npass):
        m = jnp.max(key, axis=1, keepdims=True)
        mi = m.astype(jnp.int32)
        idxs.append((2 ** idx_bits - 1) - (mi & (2 ** idx_bits - 1)))
        key = jnp.where(key == m, -1.0, key)
    return idxs


def _select_topk_general(d, col, nvalid, npass):
    d = jnp.where(col < nvalid, d, -1.0)
    idxs = []
    for _ in range(npass):
        m = jnp.max(d, axis=1, keepdims=True)
        idx = jnp.min(jnp.where(d == m, col, BIG), axis=1, keepdims=True)
        idxs.append(idx)
        d = jnp.where(col == idx, -1.0, d)
    return idxs


def _fused_body(fast, cells_ref, coords_ref, cellsT_ref,
                aic_ref, nbc_ref, cfa_ref, best_ref, bidx_ref):
    i = pl.program_id(0)
    cx = cells_ref[:, 0:1]
    cy = cells_ref[:, 1:2]
    cz = cells_ref[:, 2:3]
    if fast:
        x = coords_ref[0:1, :]
        y = coords_ref[1:2, :]
        z = coords_ref[2:3, :]
        an = x * x + y * y + z * z
        cn = cx * cx + cy * cy + cz * cz
        prod = lax.dot_general(cells_ref[:, 0:3], coords_ref[...],
                               (((1,), (0,)), ((), ())),
                               preferred_element_type=jnp.float32)
        d = cn + an - 2.0 * prod
    else:
        dx = coords_ref[0:1, :] - cx
        dy = coords_ref[1:2, :] - cy
        dz = coords_ref[2:3, :] - cz
        d = dx * dx + dy * dy + dz * dz

    if fast:
        rowf = lax.broadcasted_iota(jnp.int32, (ROWS, NPAD), 0).astype(jnp.float32)
        rowk = d * 32768.0 + (rowf + float(ROWS) * i.astype(jnp.float32))
        rk = jnp.min(rowk, axis=0, keepdims=True)

        @pl.when(i == 0)
        def _init():
            best_ref[...] = jnp.full((1, NPAD), 3e38, jnp.float32)

        best_ref[...] = jnp.minimum(best_ref[...], rk)

        @pl.when(i == pl.num_programs(0) - 1)
        def _fin():
            cfa_ref[...] = best_ref[...].astype(jnp.int32) & 32767
    else:
        @pl.when(i == 0)
        def _init():
            best_ref[...] = jnp.full((1, NPAD), jnp.inf, jnp.float32)
            bidx_ref[...] = jnp.zeros((1, NPAD), jnp.int32)

        best = best_ref[...]
        bidx = bidx_ref[...]
        for r in range(ROWS):
            dr = d[r:r + 1, :]
            upd = dr < best
            best = jnp.where(upd, dr, best)
            bidx = jnp.where(upd, i * ROWS + r, bidx)
        best_ref[...] = best
        bidx_ref[...] = bidx

        @pl.when(i == pl.num_programs(0) - 1)
        def _fin():
            cfa_ref[...] = bidx_ref[...]

    col = lax.broadcasted_iota(jnp.int32, (ROWS, NPAD), 1)
    if fast:
        idxs = _select_topk_fast(d, col, N, K, 15)
    else:
        idxs = _select_topk_general(d, col, N, K)
    aic_ref[...] = jnp.concatenate(idxs, axis=1)

    dx3 = cellsT_ref[0:1, :] - cx
    dy3 = cellsT_ref[1:2, :] - cy
    dz3 = cellsT_ref[2:3, :] - cz
    d3 = dx3 * dx3 + dy3 * dy3 + dz3 * dz3
    col3 = lax.broadcasted_iota(jnp.int32, (ROWS, CCOLS), 1)
    if fast:
        idxs3 = _select_topk_fast(d3, col3, NCELL, NNB, 10)
    else:
        idxs3 = _select_topk_general(d3, col3, NCELL, NNB)
    idxs3.append(jnp.zeros((ROWS, 32 - NNB), jnp.int32))
    nbc_ref[...] = jnp.concatenate(idxs3, axis=1)


def _sc_body(pc_hbm, aic_hbm, nbr_hbm, cfa_hbm, out_hbm,
             pv, aicv, nbrv, cfav, outv, sem):
    wid = lax.axis_index("s") * 2 + lax.axis_index("c")
    base = wid * APT
    copies = [pltpu.async_copy(pc_hbm, pv, sem),
              pltpu.async_copy(aic_hbm, aicv, sem),
              pltpu.async_copy(nbr_hbm, nbrv, sem),
              pltpu.async_copy(cfa_hbm.at[pl.ds(base, APT)], cfav, sem)]
    for c in copies:
        c.wait()

    lane = lax.iota(jnp.int32, 16)

    def merge(ka, va, kb, vb):
        rkb = lax.rev(kb, (0,))
        rvb = lax.rev(vb, (0,))
        take = ka >= rkb
        km = jnp.where(take, ka, rkb)
        vm = jnp.where(take, va, rvb)
        return plsc.sort_key_val(km, vm, descending=True)

    @plsc.parallel_loop(0, APT, unroll=16)
    def body(i):
        iv = jnp.full((16,), i, jnp.int32)
        cid = plsc.load_gather(cfav, [iv])
        av = iv + base
        pa = plsc.load_gather(pv, [av])
        xa = pa >> 10
        ya = (pa >> 5) & 31
        za = pa & 31
        ks, vs = [], []
        for v in range(NVREG):
            t = lane + (v * 16)
            cslot = t >> 3
            w = t & 7
            nb = plsc.load_gather(nbrv, [cid * 32 + cslot])
            cand = plsc.load_gather(aicv, [nb * K + w])
            pc = plsc.load_gather(pv, [cand])
            dx = xa - (pc >> 10)
            dy = ya - ((pc >> 5) & 31)
            dz = za - (pc & 31)
            di = dx * dx + dy * dy + dz * dz
            key = di * 256 + (255 - t)
            sk, sv = plsc.sort_key_val(key, cand, descending=True)
            ks.append(sk)
            vs.append(sv)
        while len(ks) > 1:
            nk, nv = [], []
            for j in range(0, len(ks) - 1, 2):
                k2, v2 = merge(ks[j], vs[j], ks[j + 1], vs[j + 1])
                nk.append(k2)
                nv.append(v2)
            if len(ks) % 2:
                nk.append(ks[-1])
                nv.append(vs[-1])
            ks, vs = nk, nv
        outv[pl.ds(i * M, M)] = vs[0]

    tail = (N - (NW - 1) * APT) * M

    @pl.when(wid < NW - 1)
    def _full():
        pltpu.sync_copy(outv, out_hbm.at[pl.ds(base * M, APT * M)])

    @pl.when(wid == NW - 1)
    def _part():
        pltpu.sync_copy(outv.at[pl.ds(0, tail)],
                        out_hbm.at[pl.ds((NW - 1) * APT * M, tail)])


def _grid_cells(start, stop):
    step = (stop - start).astype(jnp.float32) / jnp.float32(NSIDE)
    r = start.astype(jnp.float32) + jnp.arange(NSIDE, dtype=jnp.float32) * step
    mesh = jnp.stack(jnp.meshgrid(*([r] * 3)))
    return jnp.transpose(mesh).reshape(NCELL, 3)


@jax.jit
def kernel(coords):
    start = jnp.min(coords).astype(jnp.int32)
    stop = jnp.max(coords).astype(jnp.int32)
    cells = _grid_cells(start, stop)

    cells_pad = jnp.full((CPAD, 128), 1e9, jnp.float32).at[:NCELL, :3].set(cells)
    cells_t = jnp.full((8, CCOLS), 1e9, jnp.float32).at[:3, :NCELL].set(cells.T)
    ct = jnp.zeros((3, NPAD), jnp.float32).at[:, :N].set(coords.T)

    def run_fused(fast):
        def go(_):
            return pl.pallas_call(
                functools.partial(_fused_body, fast),
                grid=(CPAD // ROWS,),
                in_specs=[pl.BlockSpec((ROWS, 128), lambda i: (i, 0)),
                          pl.BlockSpec((3, NPAD), lambda i: (0, 0)),
                          pl.BlockSpec((8, CCOLS), lambda i: (0, 0))],
                out_specs=[pl.BlockSpec((ROWS, K), lambda i: (i, 0)),
                           pl.BlockSpec((ROWS, 32), lambda i: (i, 0)),
                           pl.BlockSpec((1, NPAD), lambda i: (0, 0))],
                out_shape=[jax.ShapeDtypeStruct((CPAD, K), jnp.int32),
                           jax.ShapeDtypeStruct((CPAD, 32), jnp.int32),
                           jax.ShapeDtypeStruct((1, NPAD), jnp.int32)],
                scratch_shapes=[pltpu.VMEM((1, NPAD), jnp.float32),
                                pltpu.VMEM((1, NPAD), jnp.int32)],
            )(cells_pad, ct, cells_t)
        return go

    span = stop - start
    aic, nbc, cfa = lax.cond((span == 9) | (span == 0),
                             run_fused(True), run_fused(False), coords)

    sc = pl.kernel(
        _sc_body,
        out_type=jax.ShapeDtypeStruct((N * M,), jnp.int32),
        mesh=plsc.VectorSubcoreMesh(core_axis_name="c", subcore_axis_name="s",
                                    num_cores=2, num_subcores=16),
        compiler_params=pltpu.CompilerParams(needs_layout_passes=False),
        scratch_types=[
            pltpu.VMEM((NPAD,), jnp.int32),
            pltpu.VMEM((CPAD * K,), jnp.int32),
            pltpu.VMEM((CPAD * 32,), jnp.int32),
            pltpu.VMEM((APT,), jnp.int32),
            pltpu.VMEM((APT * M,), jnp.int32),
            pltpu.SemaphoreType.DMA,
        ],
    )
    ci = ct.astype(jnp.int32)
    pxyz = ci[0] * 1024 + ci[1] * 32 + ci[2]
    out = sc(pxyz, aic.reshape(CPAD * K), nbc.reshape(CPAD * 32),
             cfa.reshape(NPAD))
    return out.reshape(N, M)


# speedup vs baseline: 1.2653x; 1.2653x over previous
import functools

import jax
import jax.numpy as jnp
from jax import lax
from jax.experimental import pallas as pl
from jax.experimental.pallas import tpu as pltpu
from jax.experimental.pallas import tpu_sc as plsc

N = 20000
NPAD = 20480
NCELL = 729
CPAD = 768
CCOLS = 768
K = 8
M = 16
NNB = 26
NSIDE = 9
NVREG = (NNB * K) // 16
BIG = 1 << 30
NW = 32
APT = NPAD // NW


ROWS = 96


def _select_topk_fast(d, col, nvalid, npass, idx_bits):
    half = float(2 ** idx_bits)
    key = jnp.where(col < nvalid,
                    d * half + ((half - 1.0) - col.astype(jnp.float32)),
                    -1.0)
    idxs = []
    for _ in range(npass):
        m = jnp.max(key, axis=1, keepdims=True)
        mi = m.astype(jnp.int32)
        idxs.append((2 ** idx_bits - 1) - (mi & (2 ** idx_bits - 1)))
        key = jnp.where(key == m, -1.0, key)
    return idxs


def _select_topk_general(d, col, nvalid, npass):
    d = jnp.where(col < nvalid, d, -1.0)
    idxs = []
    for _ in range(npass):
        m = jnp.max(d, axis=1, keepdims=True)
        idx = jnp.min(jnp.where(d == m, col, BIG), axis=1, keepdims=True)
        idxs.append(idx)
        d = jnp.where(col == idx, -1.0, d)
    return idxs


def _fused_body(fast, cells_ref, coords_ref, cellsT_ref,
                aic_ref, nbc_ref, cfa_ref, best_ref, bidx_ref):
    i = pl.program_id(0)
    cx = cells_ref[:, 0:1]
    cy = cells_ref[:, 1:2]
    cz = cells_ref[:, 2:3]
    if fast:
        x = coords_ref[0:1, :]
        y = coords_ref[1:2, :]
        z = coords_ref[2:3, :]
        an = x * x + y * y + z * z
        cn = cx * cx + cy * cy + cz * cz
        prod = lax.dot_general(cells_ref[:, 0:3], coords_ref[...],
                               (((1,), (0,)), ((), ())),
                               preferred_element_type=jnp.float32)
        d = cn + an - 2.0 * prod
    else:
        dx = coords_ref[0:1, :] - cx
        dy = coords_ref[1:2, :] - cy
        dz = coords_ref[2:3, :] - cz
        d = dx * dx + dy * dy + dz * dz

    if fast:
        rowf = lax.broadcasted_iota(jnp.int32, (ROWS, NPAD), 0).astype(jnp.float32)
        rowk = d * 32768.0 + (rowf + float(ROWS) * i.astype(jnp.float32))
        rk = jnp.min(rowk, axis=0, keepdims=True)

        @pl.when(i == 0)
        def _init():
            best_ref[...] = jnp.full((1, NPAD), 3e38, jnp.float32)

        best_ref[...] = jnp.minimum(best_ref[...], rk)

        @pl.when(i == pl.num_programs(0) - 1)
        def _fin():
            cfa_ref[...] = best_ref[...].astype(jnp.int32) & 32767
    else:
        @pl.when(i == 0)
        def _init():
            best_ref[...] = jnp.full((1, NPAD), jnp.inf, jnp.float32)
            bidx_ref[...] = jnp.zeros((1, NPAD), jnp.int32)

        best = best_ref[...]
        bidx = bidx_ref[...]
        for r in range(ROWS):
            dr = d[r:r + 1, :]
            upd = dr < best
            best = jnp.where(upd, dr, best)
            bidx = jnp.where(upd, i * ROWS + r, bidx)
        best_ref[...] = best
        bidx_ref[...] = bidx

        @pl.when(i == pl.num_programs(0) - 1)
        def _fin():
            cfa_ref[...] = bidx_ref[...]

    col = lax.broadcasted_iota(jnp.int32, (ROWS, NPAD), 1)
    if fast:
        idxs = _select_topk_fast(d, col, N, K, 15)
    else:
        idxs = _select_topk_general(d, col, N, K)
    aic_ref[...] = jnp.concatenate(idxs, axis=1)

    dx3 = cellsT_ref[0:1, :] - cx
    dy3 = cellsT_ref[1:2, :] - cy
    dz3 = cellsT_ref[2:3, :] - cz
    d3 = dx3 * dx3 + dy3 * dy3 + dz3 * dz3
    col3 = lax.broadcasted_iota(jnp.int32, (ROWS, CCOLS), 1)
    if fast:
        idxs3 = _select_topk_fast(d3, col3, NCELL, NNB, 10)
    else:
        idxs3 = _select_topk_general(d3, col3, NCELL, NNB)
    idxs3.append(jnp.zeros((ROWS, 32 - NNB), jnp.int32))
    nbc_ref[...] = jnp.concatenate(idxs3, axis=1)


def _sc_body(pc_hbm, aic_hbm, nbr_hbm, cfa_hbm, out_hbm,
             pv, aicv, nbrv, cfav, outv, sem):
    wid = lax.axis_index("s") * 2 + lax.axis_index("c")
    base = wid * APT
    copies = [pltpu.async_copy(pc_hbm, pv, sem),
              pltpu.async_copy(aic_hbm, aicv, sem),
              pltpu.async_copy(nbr_hbm, nbrv, sem),
              pltpu.async_copy(cfa_hbm.at[pl.ds(base, APT)], cfav, sem)]
    for c in copies:
        c.wait()

    lane = lax.iota(jnp.int32, 16)

    def merge(ka, va, kb, vb):
        rkb = lax.rev(kb, (0,))
        rvb = lax.rev(vb, (0,))
        take = ka >= rkb
        km = jnp.where(take, ka, rkb)
        vm = jnp.where(take, va, rvb)
        return plsc.sort_key_val(km, vm, descending=True)

    @plsc.parallel_loop(0, APT, unroll=8)
    def body(i):
        iv = jnp.full((16,), i, jnp.int32)
        cid = plsc.load_gather(cfav, [iv])
        av = iv + base
        pa = plsc.load_gather(pv, [av])
        xa = pa >> 10
        ya = (pa >> 5) & 31
        za = pa & 31
        ks, vs = [], []
        for v in range(NVREG):
            t = lane + (v * 16)
            cslot = t >> 3
            w = t & 7
            nb = plsc.load_gather(nbrv, [cid * 32 + cslot])
            cand = plsc.load_gather(aicv, [nb * K + w])
            pc = plsc.load_gather(pv, [cand])
            dx = xa - (pc >> 10)
            dy = ya - ((pc >> 5) & 31)
            dz = za - (pc & 31)
            di = dx * dx + dy * dy + dz * dz
            key = di * 256 + (255 - t)
            sk, sv = plsc.sort_key_val(key, cand, descending=True)
            ks.append(sk)
            vs.append(sv)
        while len(ks) > 1:
            nk, nv = [], []
            for j in range(0, len(ks) - 1, 2):
                k2, v2 = merge(ks[j], vs[j], ks[j + 1], vs[j + 1])
                nk.append(k2)
                nv.append(v2)
            if len(ks) % 2:
                nk.append(ks[-1])
                nv.append(vs[-1])
            ks, vs = nk, nv
        outv[pl.ds(i * M, M)] = vs[0]

    tail = (N - (NW - 1) * APT) * M

    @pl.when(wid < NW - 1)
    def _full():
        pltpu.sync_copy(outv, out_hbm.at[pl.ds(base * M, APT * M)])

    @pl.when(wid == NW - 1)
    def _part():
        pltpu.sync_copy(outv.at[pl.ds(0, tail)],
                        out_hbm.at[pl.ds((NW - 1) * APT * M, tail)])


def _grid_cells(start, stop):
    step = (stop - start).astype(jnp.float32) / jnp.float32(NSIDE)
    r = start.astype(jnp.float32) + jnp.arange(NSIDE, dtype=jnp.float32) * step
    mesh = jnp.stack(jnp.meshgrid(*([r] * 3)))
    return jnp.transpose(mesh).reshape(NCELL, 3)


@jax.jit
def kernel(coords):
    start = jnp.min(coords).astype(jnp.int32)
    stop = jnp.max(coords).astype(jnp.int32)
    cells = _grid_cells(start, stop)

    cells_pad = jnp.full((CPAD, 128), 1e9, jnp.float32).at[:NCELL, :3].set(cells)
    cells_t = jnp.full((8, CCOLS), 1e9, jnp.float32).at[:3, :NCELL].set(cells.T)
    ct = jnp.zeros((3, NPAD), jnp.float32).at[:, :N].set(coords.T)

    def run_fused(fast):
        def go(_):
            return pl.pallas_call(
                functools.partial(_fused_body, fast),
                grid=(CPAD // ROWS,),
                in_specs=[pl.BlockSpec((ROWS, 128), lambda i: (i, 0)),
                          pl.BlockSpec((3, NPAD), lambda i: (0, 0)),
                          pl.BlockSpec((8, CCOLS), lambda i: (0, 0))],
                out_specs=[pl.BlockSpec((ROWS, K), lambda i: (i, 0)),
                           pl.BlockSpec((ROWS, 32), lambda i: (i, 0)),
                           pl.BlockSpec((1, NPAD), lambda i: (0, 0))],
                out_shape=[jax.ShapeDtypeStruct((CPAD, K), jnp.int32),
                           jax.ShapeDtypeStruct((CPAD, 32), jnp.int32),
                           jax.ShapeDtypeStruct((1, NPAD), jnp.int32)],
                scratch_shapes=[pltpu.VMEM((1, NPAD), jnp.float32),
                                pltpu.VMEM((1, NPAD), jnp.int32)],
            )(cells_pad, ct, cells_t)
        return go

    span = stop - start
    aic, nbc, cfa = lax.cond((span == 9) | (span == 0),
                             run_fused(True), run_fused(False), coords)

    sc = pl.kernel(
        _sc_body,
        out_type=jax.ShapeDtypeStruct((N * M,), jnp.int32),
        mesh=plsc.VectorSubcoreMesh(core_axis_name="c", subcore_axis_name="s",
                                    num_cores=2, num_subcores=16),
        compiler_params=pltpu.CompilerParams(needs_layout_passes=False),
        scratch_types=[
            pltpu.VMEM((NPAD,), jnp.int32),
            pltpu.VMEM((CPAD * K,), jnp.int32),
            pltpu.VMEM((CPAD * 32,), jnp.int32),
            pltpu.VMEM((APT,), jnp.int32),
            pltpu.VMEM((APT * M,), jnp.int32),
            pltpu.SemaphoreType.DMA,
        ],
    )
    ci = ct.astype(jnp.int32)
    pxyz = ci[0] * 1024 + ci[1] * 32 + ci[2]
    out = sc(pxyz, aic.reshape(CPAD * K), nbc.reshape(CPAD * 32),
             cfa.reshape(NPAD))
    return out.reshape(N, M)
